# baseline (speedup 1.0000x reference)
import functools

import jax
import jax.numpy as jnp
from jax import lax
from jax.experimental import pallas as pl
from jax.experimental.pallas import tpu as pltpu
from jax.experimental.pallas import tpu_sc as plsc

D_MODEL = 128
NUM_CORES = 2
NUM_SUBCORES = 16
NUM_WORKERS = NUM_CORES * NUM_SUBCORES
CHUNK = 128
NBUF = 5
GDEPTH = 4
WDEPTH = 1


def _emb_body(x_hbm, table_hbm, out_hbm, idx_v, rows_v, gsem, wsem):
    n_chunks = x_hbm.shape[1]
    wid = lax.axis_index("s") * NUM_CORES + lax.axis_index("c")
    base = wid * n_chunks * CHUNK
    pltpu.sync_copy(x_hbm.at[wid], idx_v)

    def gather(c, b):
        pltpu.async_copy(table_hbm.at[idx_v.at[c]], rows_v.at[b], gsem.at[b])

    def write_desc(c, b):
        out_slab = out_hbm.at[pl.ds(base + c * CHUNK, CHUNK)]
        return pltpu.make_async_copy(rows_v.at[b], out_slab, wsem.at[b])

    for c in range(GDEPTH):
        gather(c, c)

    def group_body(g, _):
        for b in range(NBUF):
            c = g * NBUF + b
            pltpu.make_async_copy(
                table_hbm.at[idx_v.at[c]], rows_v.at[b], gsem.at[b]
            ).wait()
            write_desc(c, b).start()

            @pl.when(c >= WDEPTH)
            def _():
                write_desc(c - WDEPTH, (b - WDEPTH) % NBUF).wait()

            @pl.when(c + GDEPTH < n_chunks)
            def _():
                gather(c + GDEPTH, (b + GDEPTH) % NBUF)

        return 0

    lax.fori_loop(0, n_chunks // NBUF, group_body, 0)

    for k in range(WDEPTH):
        c = n_chunks - WDEPTH + k
        write_desc(c, c % NBUF).wait()


def kernel(x, table):
    b, t = x.shape
    total = b * t
    assert total % (NUM_WORKERS * CHUNK * NBUF) == 0
    n_chunks = total // (NUM_WORKERS * CHUNK)
    x_blocks = x.reshape(NUM_WORKERS, n_chunks, CHUNK)

    emb = functools.partial(
        pl.kernel,
        out_type=jax.ShapeDtypeStruct((total, D_MODEL), jnp.float32),
        mesh=plsc.VectorSubcoreMesh(core_axis_name="c", subcore_axis_name="s"),
        scratch_types=[
            pltpu.VMEM((n_chunks, CHUNK), jnp.int32),
            pltpu.VMEM((NBUF, CHUNK, D_MODEL), jnp.float32),
            pltpu.SemaphoreType.DMA((NBUF,)),
            pltpu.SemaphoreType.DMA((NBUF,)),
        ],
    )(_emb_body)

    out = emb(x_blocks, table)
    return out.reshape(b, t, D_MODEL)


# speedup vs baseline: 1.0039x; 1.0039x over previous
import functools

import jax
import jax.numpy as jnp
from jax import lax
from jax.experimental import pallas as pl
from jax.experimental.pallas import tpu as pltpu
from jax.experimental.pallas import tpu_sc as plsc

D_MODEL = 128
NUM_CORES = 2
NUM_SUBCORES = 16
NUM_WORKERS = NUM_CORES * NUM_SUBCORES
CHUNK = 128
NBUF = 5


def _emb_body(x_hbm, table_hbm, out_hbm, idx_v, rows_v, gsem, wsem):
    n_chunks = x_hbm.shape[1]
    wid = lax.axis_index("s") * NUM_CORES + lax.axis_index("c")
    base = wid * n_chunks * CHUNK
    pltpu.sync_copy(x_hbm.at[wid], idx_v)

    def gather(c, b):
        pltpu.async_copy(table_hbm.at[idx_v.at[c]], rows_v.at[b], gsem.at[b])

    for b in range(NBUF):
        gather(b, b)

    def group_body(g, _):
        for b in range(NBUF):
            c = g * NBUF + b
            pltpu.make_async_copy(
                table_hbm.at[idx_v.at[c]], rows_v.at[b], gsem.at[b]
            ).wait()
            out_slab = out_hbm.at[pl.ds(base + c * CHUNK, CHUNK)]
            pltpu.async_copy(rows_v.at[b], out_slab, wsem.at[b])
            pltpu.make_async_copy(rows_v.at[b], out_slab, wsem.at[b]).wait()

            @pl.when(c + NBUF < n_chunks)
            def _():
                gather(c + NBUF, b)

        return 0

    lax.fori_loop(0, n_chunks // NBUF, group_body, 0)


def kernel(x, table):
    b, t = x.shape
    total = b * t
    assert total % (NUM_WORKERS * CHUNK * NBUF) == 0
    n_chunks = total // (NUM_WORKERS * CHUNK)
    x_blocks = x.reshape(NUM_WORKERS, n_chunks, CHUNK)

    emb = functools.partial(
        pl.kernel,
        out_type=jax.ShapeDtypeStruct((total, D_MODEL), jnp.float32),
        mesh=plsc.VectorSubcoreMesh(core_axis_name="c", subcore_axis_name="s"),
        scratch_types=[
            pltpu.VMEM((n_chunks, CHUNK), jnp.int32),
            pltpu.VMEM((NBUF, CHUNK, D_MODEL), jnp.float32),
            pltpu.SemaphoreType.DMA((NBUF,)),
            pltpu.SemaphoreType.DMA((NBUF,)),
        ],
    )(_emb_body)

    out = emb(x_blocks, table)
    return out.reshape(b, t, D_MODEL)
